# baseline (speedup 1.0000x reference)
import functools

import jax
import jax.numpy as jnp
from jax import lax
from jax.experimental import pallas as pl
from jax.experimental.pallas import tpu as pltpu
from jax.experimental.pallas import tpu_sc as plsc

K = 2
TILE = 256
BH = 2048
BD = 1024
CHD = 256


def _router_body(x_ref, rw_ref, wf_ref):
    logits = jax.lax.dot_general(
        x_ref[...], rw_ref[...], (((1,), (1,)), ((), ())),
        preferred_element_type=jnp.float32)
    m = jnp.max(logits, axis=1, keepdims=True)
    p = jnp.exp(logits - m)
    p = p / jnp.sum(p, axis=1, keepdims=True)
    e_dim = logits.shape[1]
    li = jax.lax.broadcasted_iota(jnp.int32, p.shape, 1)
    m1 = jnp.max(p, axis=1, keepdims=True)
    i1 = jnp.min(jnp.where(p >= m1, li, e_dim), axis=1, keepdims=True)
    first1 = li == i1
    p2 = jnp.where(first1, -1.0, p)
    m2 = jnp.max(p2, axis=1, keepdims=True)
    i2 = jnp.min(jnp.where(p2 >= m2, li, e_dim), axis=1, keepdims=True)
    first2 = li == i2
    wsum = m1 + m2
    wf_ref[...] = jnp.where(first1 | first2, p / wsum, 0.0)


def _kd_body(w_ref, pos2_ref, wv_ref, te_ref, tv_ref,
             cnt_ref, carry_ref, pb_ref):
    ph = pl.program_id(0)
    c = pl.program_id(1)
    nt = te_ref.shape[1]
    w = w_ref[...]
    e_dim = w.shape[1]
    chosen = (w > 0.0).astype(jnp.float32)

    @pl.when((ph == 0) & (c == 0))
    def _init():
        cnt_ref[...] = jnp.zeros_like(cnt_ref)

    @pl.when(ph == 0)
    def _count():
        cnt_ref[...] += jnp.sum(chosen, axis=0, keepdims=True)

    @pl.when((ph == 1) & (c == 0))
    def _base():
        counts = cnt_ref[...]
        tiles_pe = jnp.floor((counts + (TILE - 1)) * (1.0 / TILE))
        r8 = jax.lax.broadcasted_iota(jnp.int32, (e_dim, e_dim), 0)
        c8 = jax.lax.broadcasted_iota(jnp.int32, (e_dim, e_dim), 1)
        ut = (r8 <= c8).astype(jnp.float32)
        tile_cum = jax.lax.dot_general(
            tiles_pe, ut, (((1,), (0,)), ((), ())),
            preferred_element_type=jnp.float32)
        pb_ref[...] = (tile_cum - tiles_pe) * TILE
        carry_ref[...] = jnp.zeros_like(carry_ref)
        pad = jnp.full((1, nt - e_dim), 32768.0, jnp.float32)
        cum_pad = jnp.concatenate([tile_cum, pad], axis=1)
        rn = jax.lax.broadcasted_iota(jnp.int32, (nt, nt), 0)
        cn = jax.lax.broadcasted_iota(jnp.int32, (nt, nt), 1)
        ident = (rn == cn).astype(jnp.float32)
        cum_sub = jax.lax.dot_general(
            ident, cum_pad, (((1,), (1,)), ((), ())),
            preferred_element_type=jnp.float32)
        tidx = jax.lax.broadcasted_iota(jnp.int32, (nt, nt), 1
                                        ).astype(jnp.float32)
        a = (cum_sub <= tidx).astype(jnp.float32)
        ones_row = (rn == rn).astype(jnp.float32)[:1]
        te_row = jax.lax.dot_general(
            ones_row, a, (((1,), (0,)), ((), ())),
            preferred_element_type=jnp.float32)
        te_ref[...] = jnp.minimum(te_row, e_dim - 1).astype(jnp.int32)
        total = jnp.max(tile_cum, axis=1, keepdims=True)
        ti = jax.lax.broadcasted_iota(jnp.int32, (1, nt), 1
                                      ).astype(jnp.float32)
        tv_ref[...] = (ti < total).astype(jnp.int32)

    @pl.when(ph == 1)
    def _pos():
        n = w.shape[0]
        r = jax.lax.broadcasted_iota(jnp.int32, (n, n), 0)
        cc = jax.lax.broadcasted_iota(jnp.int32, (n, n), 1)
        lt = (cc < r).astype(jnp.float32)
        csum_ex = jax.lax.dot_general(
            lt, chosen, (((1,), (0,)), ((), ())),
            preferred_element_type=jnp.float32)
        rank = carry_ref[...] + csum_ex
        pos = pb_ref[...] + rank
        carry_ref[...] += jnp.sum(chosen, axis=0, keepdims=True)
        li = jax.lax.broadcasted_iota(jnp.int32, (n, e_dim), 1)
        i1 = jnp.min(jnp.where(w > 0.0, li, e_dim), axis=1, keepdims=True)
        first1 = li == i1
        second = (w > 0.0) & jnp.logical_not(first1)
        p0 = jnp.sum(jnp.where(first1, pos, 0.0), axis=1, keepdims=True)
        p1 = jnp.sum(jnp.where(second, pos, 0.0), axis=1, keepdims=True)
        pos2_ref[...] = jnp.concatenate([p0, p1], axis=1).astype(jnp.int32)
        w0 = jnp.sum(jnp.where(first1, w, 0.0), axis=1, keepdims=True)
        w1 = jnp.sum(jnp.where(second, w, 0.0), axis=1, keepdims=True)
        wv_ref[...] = jnp.concatenate([w0, w1], axis=1)


def _make_ks(t_num, nslot, ntot, dim):
    info = plsc.get_sparse_core_info()
    nw = info.num_cores * info.num_subcores
    spw = nslot // nw
    ch = 32
    nch = spw // ch
    mesh = plsc.VectorSubcoreMesh(core_axis_name="c", subcore_axis_name="s")

    @functools.partial(
        pl.kernel, mesh=mesh,
        out_type=jax.ShapeDtypeStruct((ntot, dim), jnp.float32),
        scratch_types=[
            pltpu.VMEM((ch,), jnp.int32),
            pltpu.VMEM((ch,), jnp.int32),
            pltpu.VMEM((ch, dim), jnp.float32),
            pltpu.SemaphoreType.DMA,
        ],
    )
    def ks(flat_hbm, pos_hbm, xg_hbm, posv, tokv, rows, sem):
        wid = lax.axis_index("s") * info.num_cores + lax.axis_index("c")
        base = wid * spw
        for c in range(nch):
            off = base + c * ch
            pltpu.sync_copy(pos_hbm.at[pl.ds(off, ch)], posv)
            for j in range(ch // 16):
                s16 = lax.iota(jnp.int32, 16) + (off + j * 16)
                tokv[pl.ds(j * 16, 16)] = lax.shift_right_logical(s16, 1)
            pltpu.async_copy(flat_hbm.at[tokv], rows, sem).wait()
            pltpu.async_copy(rows, xg_hbm.at[posv], sem).wait()

    return ks


def _k1_body(te_ref, tv_ref, xg_ref, wg_ref, wu_ref, act_ref):
    t = pl.program_id(1)

    @pl.when(tv_ref[t] != 0)
    def _():
        xb = xg_ref[...]
        g = jax.lax.dot_general(xb, wg_ref[0], (((1,), (1,)), ((), ())),
                                preferred_element_type=jnp.float32)
        u = jax.lax.dot_general(xb, wu_ref[0], (((1,), (1,)), ((), ())),
                                preferred_element_type=jnp.float32)
        act_ref[...] = (g * jax.nn.sigmoid(g) * u
                        ).astype(jnp.bfloat16)


def _k2_body(te_ref, tv_ref, act_ref, wd_ref, out_ref):
    t = pl.program_id(1)

    @pl.when(tv_ref[t] != 0)
    def _():
        out_ref[...] = jax.lax.dot_general(
            act_ref[...].astype(jnp.float32), wd_ref[0],
            (((1,), (1,)), ((), ())),
            preferred_element_type=jnp.float32)


def kernel(x, router_W, Wg, Wu, Wd):
    bsz, seq, dim = x.shape
    e_num = router_W.shape[0]
    hid = Wg.shape[1]
    t_num = bsz * seq
    nslot = t_num * K
    nt = nslot // TILE + e_num
    ntot = nt * TILE
    nh = hid // BH
    nd = dim // BD

    flat = x.reshape(t_num, dim)

    bt = 512
    w_full = pl.pallas_call(
        _router_body,
        grid=(t_num // bt,),
        in_specs=[
            pl.BlockSpec((bt, dim), lambda i: (i, 0)),
            pl.BlockSpec((e_num, dim), lambda i: (0, 0)),
        ],
        out_specs=pl.BlockSpec((bt, e_num), lambda i: (i, 0)),
        out_shape=jax.ShapeDtypeStruct((t_num, e_num), jnp.float32),
    )(flat, router_W)

    nc = t_num // CHD
    pos2, wv, te2d, tv2d = pl.pallas_call(
        _kd_body,
        grid=(2, nc),
        in_specs=[pl.BlockSpec((CHD, e_num), lambda p, c: (c, 0))],
        out_specs=[
            pl.BlockSpec((CHD, K), lambda p, c: (c, 0)),
            pl.BlockSpec((CHD, K), lambda p, c: (c, 0)),
            pl.BlockSpec((1, nt), lambda p, c: (0, 0)),
            pl.BlockSpec((1, nt), lambda p, c: (0, 0)),
        ],
        out_shape=[
            jax.ShapeDtypeStruct((t_num, K), jnp.int32),
            jax.ShapeDtypeStruct((t_num, K), jnp.float32),
            jax.ShapeDtypeStruct((1, nt), jnp.int32),
            jax.ShapeDtypeStruct((1, nt), jnp.int32),
        ],
        scratch_shapes=[
            pltpu.VMEM((1, e_num), jnp.float32),
            pltpu.VMEM((1, e_num), jnp.float32),
            pltpu.VMEM((1, e_num), jnp.float32),
        ],
        compiler_params=pltpu.CompilerParams(
            dimension_semantics=("arbitrary", "arbitrary")),
    )(w_full)
    tile_e = te2d.reshape(nt)
    tile_valid = tv2d.reshape(nt)

    xg = _make_ks(t_num, nslot, ntot, dim)(flat, pos2.reshape(nslot))

    act = pl.pallas_call(
        _k1_body,
        grid_spec=pltpu.PrefetchScalarGridSpec(
            num_scalar_prefetch=2,
            grid=(nh, nt),
            in_specs=[
                pl.BlockSpec((TILE, dim), lambda h, t, te, tv: (t, 0)),
                pl.BlockSpec((1, BH, dim),
                             lambda h, t, te, tv: (te[t], h, 0)),
                pl.BlockSpec((1, BH, dim),
                             lambda h, t, te, tv: (te[t], h, 0)),
            ],
            out_specs=pl.BlockSpec((TILE, BH), lambda h, t, te, tv: (t, h)),
        ),
        out_shape=jax.ShapeDtypeStruct((ntot, hid), jnp.bfloat16),
        compiler_params=pltpu.CompilerParams(
            dimension_semantics=("arbitrary", "arbitrary")),
    )(tile_e, tile_valid, xg, Wg, Wu)

    outg = pl.pallas_call(
        _k2_body,
        grid_spec=pltpu.PrefetchScalarGridSpec(
            num_scalar_prefetch=2,
            grid=(nd, nt),
            in_specs=[
                pl.BlockSpec((TILE, hid), lambda d, t, te, tv: (t, 0)),
                pl.BlockSpec((1, BD, hid),
                             lambda d, t, te, tv: (te[t], d, 0)),
            ],
            out_specs=pl.BlockSpec((TILE, BD), lambda d, t, te, tv: (t, d)),
        ),
        out_shape=jax.ShapeDtypeStruct((ntot, dim), jnp.float32),
        compiler_params=pltpu.CompilerParams(
            dimension_semantics=("arbitrary", "arbitrary")),
    )(tile_e, tile_valid, act, Wd)

    out = wv[:, :1] * outg[pos2[:, 0]] + wv[:, 1:] * outg[pos2[:, 1]]
    return out.reshape(bsz, seq, dim)


# speedup vs baseline: 1.0998x; 1.0998x over previous
import functools

import jax
import jax.numpy as jnp
from jax import lax
from jax.experimental import pallas as pl
from jax.experimental.pallas import tpu as pltpu
from jax.experimental.pallas import tpu_sc as plsc

K = 2
TILE = 512
BH = 2048
BD = 1024
CHD = 256


def _kd_body(x_ref, rw_ref, pos2_ref, wv_ref, te_ref, tv_ref,
             cnt_ref, carry_ref, pb_ref, wfull_ref):
    ph = pl.program_id(0)
    c = pl.program_id(1)
    nt = te_ref.shape[1]
    chd = x_ref.shape[0]
    e_dim = rw_ref.shape[0]

    @pl.when((ph == 0) & (c == 0))
    def _init():
        cnt_ref[...] = jnp.zeros_like(cnt_ref)

    @pl.when(ph == 0)
    def _router():
        logits = jax.lax.dot_general(
            x_ref[...], rw_ref[...], (((1,), (1,)), ((), ())),
            preferred_element_type=jnp.float32)
        m = jnp.max(logits, axis=1, keepdims=True)
        p = jnp.exp(logits - m)
        p = p / jnp.sum(p, axis=1, keepdims=True)
        li = jax.lax.broadcasted_iota(jnp.int32, p.shape, 1)
        m1 = jnp.max(p, axis=1, keepdims=True)
        i1 = jnp.min(jnp.where(p >= m1, li, e_dim), axis=1, keepdims=True)
        first1 = li == i1
        p2 = jnp.where(first1, -1.0, p)
        m2 = jnp.max(p2, axis=1, keepdims=True)
        i2 = jnp.min(jnp.where(p2 >= m2, li, e_dim), axis=1, keepdims=True)
        first2 = li == i2
        wf = jnp.where(first1 | first2, p / (m1 + m2), 0.0)
        wfull_ref[pl.ds(c * chd, chd), :] = wf
        cnt_ref[...] += jnp.sum((wf > 0.0).astype(jnp.float32),
                                axis=0, keepdims=True)

    w = wfull_ref[pl.ds(c * chd, chd), :]
    chosen = (w > 0.0).astype(jnp.float32)

    @pl.when((ph == 1) & (c == 0))
    def _base():
        counts = cnt_ref[...]
        tiles_pe = jnp.floor((counts + (TILE - 1)) * (1.0 / TILE))
        r8 = jax.lax.broadcasted_iota(jnp.int32, (e_dim, e_dim), 0)
        c8 = jax.lax.broadcasted_iota(jnp.int32, (e_dim, e_dim), 1)
        ut = (r8 <= c8).astype(jnp.float32)
        tile_cum = jax.lax.dot_general(
            tiles_pe, ut, (((1,), (0,)), ((), ())),
            preferred_element_type=jnp.float32)
        pb_ref[...] = (tile_cum - tiles_pe) * TILE
        carry_ref[...] = jnp.zeros_like(carry_ref)
        pad = jnp.full((1, nt - e_dim), 32768.0, jnp.float32)
        cum_pad = jnp.concatenate([tile_cum, pad], axis=1)
        rn = jax.lax.broadcasted_iota(jnp.int32, (nt, nt), 0)
        cn = jax.lax.broadcasted_iota(jnp.int32, (nt, nt), 1)
        ident = (rn == cn).astype(jnp.float32)
        cum_sub = jax.lax.dot_general(
            ident, cum_pad, (((1,), (1,)), ((), ())),
            preferred_element_type=jnp.float32)
        tidx = jax.lax.broadcasted_iota(jnp.int32, (nt, nt), 1
                                        ).astype(jnp.float32)
        a = (cum_sub <= tidx).astype(jnp.float32)
        ones_row = (rn == rn).astype(jnp.float32)[:1]
        te_row = jax.lax.dot_general(
            ones_row, a, (((1,), (0,)), ((), ())),
            preferred_element_type=jnp.float32)
        te_ref[...] = jnp.minimum(te_row, e_dim - 1).astype(jnp.int32)
        total = jnp.max(tile_cum, axis=1, keepdims=True)
        ti = jax.lax.broadcasted_iota(jnp.int32, (1, nt), 1
                                      ).astype(jnp.float32)
        tv_ref[...] = (ti < total).astype(jnp.int32)

    @pl.when(ph == 1)
    def _pos():
        n = w.shape[0]
        r = jax.lax.broadcasted_iota(jnp.int32, (n, n), 0)
        cc = jax.lax.broadcasted_iota(jnp.int32, (n, n), 1)
        lt = (cc < r).astype(jnp.float32)
        csum_ex = jax.lax.dot_general(
            lt, chosen, (((1,), (0,)), ((), ())),
            preferred_element_type=jnp.float32)
        rank = carry_ref[...] + csum_ex
        pos = pb_ref[...] + rank
        carry_ref[...] += jnp.sum(chosen, axis=0, keepdims=True)
        li = jax.lax.broadcasted_iota(jnp.int32, (n, e_dim), 1)
        i1 = jnp.min(jnp.where(w > 0.0, li, e_dim), axis=1, keepdims=True)
        first1 = li == i1
        second = (w > 0.0) & jnp.logical_not(first1)
        p0 = jnp.sum(jnp.where(first1, pos, 0.0), axis=1, keepdims=True)
        p1 = jnp.sum(jnp.where(second, pos, 0.0), axis=1, keepdims=True)
        pos2_ref[...] = jnp.concatenate([p0, p1], axis=1).astype(jnp.int32)
        w0 = jnp.sum(jnp.where(first1, w, 0.0), axis=1, keepdims=True)
        w1 = jnp.sum(jnp.where(second, w, 0.0), axis=1, keepdims=True)
        wv_ref[...] = jnp.concatenate([w0, w1], axis=1)


def _make_ks(t_num, nslot, ntot, dim):
    info = plsc.get_sparse_core_info()
    nw = info.num_cores * info.num_subcores
    spw = nslot // nw
    ch = 32
    nch = spw // ch
    mesh = plsc.VectorSubcoreMesh(core_axis_name="c", subcore_axis_name="s")

    @functools.partial(
        pl.kernel, mesh=mesh,
        out_type=jax.ShapeDtypeStruct((ntot, dim), jnp.float32),
        scratch_types=[
            pltpu.VMEM((ch,), jnp.int32),
            pltpu.VMEM((ch,), jnp.int32),
            pltpu.VMEM((ch, dim), jnp.float32),
            pltpu.SemaphoreType.DMA,
        ],
    )
    def ks(flat_hbm, pos_hbm, xg_hbm, posv, tokv, rows, sem):
        wid = lax.axis_index("s") * info.num_cores + lax.axis_index("c")
        base = wid * spw
        for c in range(nch):
            off = base + c * ch
            pltpu.sync_copy(pos_hbm.at[pl.ds(off, ch)], posv)
            for j in range(ch // 16):
                s16 = lax.iota(jnp.int32, 16) + (off + j * 16)
                tokv[pl.ds(j * 16, 16)] = lax.shift_right_logical(s16, 1)
            pltpu.async_copy(flat_hbm.at[tokv], rows, sem).wait()
            pltpu.async_copy(rows, xg_hbm.at[posv], sem).wait()

    return ks


def _k1_body(te_ref, tv_ref, xg_ref, wg_ref, wu_ref, act_ref):
    t = pl.program_id(1)

    @pl.when(tv_ref[t] != 0)
    def _():
        xb = xg_ref[...]
        g = jax.lax.dot_general(xb, wg_ref[0], (((1,), (1,)), ((), ())),
                                preferred_element_type=jnp.float32)
        u = jax.lax.dot_general(xb, wu_ref[0], (((1,), (1,)), ((), ())),
                                preferred_element_type=jnp.float32)
        act_ref[...] = (g * jax.nn.sigmoid(g) * u
                        ).astype(jnp.bfloat16)


def _k2_body(te_ref, tv_ref, act_ref, wd_ref, out_ref):
    t = pl.program_id(1)

    @pl.when(tv_ref[t] != 0)
    def _():
        out_ref[...] = jax.lax.dot_general(
            act_ref[...].astype(jnp.float32), wd_ref[0],
            (((1,), (1,)), ((), ())),
            preferred_element_type=jnp.float32)


def kernel(x, router_W, Wg, Wu, Wd):
    bsz, seq, dim = x.shape
    e_num = router_W.shape[0]
    hid = Wg.shape[1]
    t_num = bsz * seq
    nslot = t_num * K
    nt = nslot // TILE + e_num
    ntot = nt * TILE
    nh = hid // BH
    nd = dim // BD

    flat = x.reshape(t_num, dim)

    nc = t_num // CHD
    pos2, wv, te2d, tv2d = pl.pallas_call(
        _kd_body,
        grid=(2, nc),
        in_specs=[
            pl.BlockSpec((CHD, dim),
                         lambda p, c: (jnp.where(p == 0, c, 0), 0)),
            pl.BlockSpec((e_num, dim), lambda p, c: (0, 0)),
        ],
        out_specs=[
            pl.BlockSpec((CHD, K), lambda p, c: (c, 0)),
            pl.BlockSpec((CHD, K), lambda p, c: (c, 0)),
            pl.BlockSpec((1, nt), lambda p, c: (0, 0)),
            pl.BlockSpec((1, nt), lambda p, c: (0, 0)),
        ],
        out_shape=[
            jax.ShapeDtypeStruct((t_num, K), jnp.int32),
            jax.ShapeDtypeStruct((t_num, K), jnp.float32),
            jax.ShapeDtypeStruct((1, nt), jnp.int32),
            jax.ShapeDtypeStruct((1, nt), jnp.int32),
        ],
        scratch_shapes=[
            pltpu.VMEM((1, e_num), jnp.float32),
            pltpu.VMEM((1, e_num), jnp.float32),
            pltpu.VMEM((1, e_num), jnp.float32),
            pltpu.VMEM((t_num, e_num), jnp.float32),
        ],
        compiler_params=pltpu.CompilerParams(
            dimension_semantics=("arbitrary", "arbitrary")),
    )(flat, router_W)
    tile_e = te2d.reshape(nt)
    tile_valid = tv2d.reshape(nt)

    xg = _make_ks(t_num, nslot, ntot, dim)(flat, pos2.reshape(nslot))

    act = pl.pallas_call(
        _k1_body,
        grid_spec=pltpu.PrefetchScalarGridSpec(
            num_scalar_prefetch=2,
            grid=(nh, nt),
            in_specs=[
                pl.BlockSpec((TILE, dim), lambda h, t, te, tv: (t, 0)),
                pl.BlockSpec((1, BH, dim),
                             lambda h, t, te, tv: (te[t], h, 0)),
                pl.BlockSpec((1, BH, dim),
                             lambda h, t, te, tv: (te[t], h, 0)),
            ],
            out_specs=pl.BlockSpec((TILE, BH), lambda h, t, te, tv: (t, h)),
        ),
        out_shape=jax.ShapeDtypeStruct((ntot, hid), jnp.bfloat16),
        compiler_params=pltpu.CompilerParams(
            dimension_semantics=("arbitrary", "arbitrary")),
    )(tile_e, tile_valid, xg, Wg, Wu)

    outg = pl.pallas_call(
        _k2_body,
        grid_spec=pltpu.PrefetchScalarGridSpec(
            num_scalar_prefetch=2,
            grid=(nd, nt),
            in_specs=[
                pl.BlockSpec((TILE, hid), lambda d, t, te, tv: (t, 0)),
                pl.BlockSpec((1, BD, hid),
                             lambda d, t, te, tv: (te[t], d, 0)),
            ],
            out_specs=pl.BlockSpec((TILE, BD), lambda d, t, te, tv: (t, d)),
        ),
        out_shape=jax.ShapeDtypeStruct((ntot, dim), jnp.float32),
        compiler_params=pltpu.CompilerParams(
            dimension_semantics=("arbitrary", "arbitrary")),
    )(tile_e, tile_valid, act, Wd)

    out = wv[:, :1] * outg[pos2[:, 0]] + wv[:, 1:] * outg[pos2[:, 1]]
    return out.reshape(bsz, seq, dim)
